# Initial kernel scaffold; baseline (speedup 1.0000x reference)
import jax
import jax.numpy as jnp
from jax.experimental import pallas as pl


def kernel(x, Wg, Wnoise, noise_eps):
    raise NotImplementedError("write your pallas kernel here")


# speedup vs baseline: 1.9480x; 1.9480x over previous
import functools

import jax
import jax.numpy as jnp
from jax import lax
from jax.experimental import pallas as pl
from jax.experimental.pallas import tpu as pltpu
from jax.experimental.pallas import tpu_sc as plsc

_N_TOK = 16384
_DIM = 2048
_NE = 64
_UTIL = 0.01

_BT = 512
_NW = 32
_TPW = _N_TOK // _NW
_L = 16


def _logits_body(x_ref, wc_ref, eps_ref, out_ref):
    both = lax.dot_general(
        x_ref[...], wc_ref[...], (((1,), (0,)), ((), ())),
        preferred_element_type=jnp.float32)
    g = both[:, :_NE]
    n = both[:, _NE:]
    sp = jnp.maximum(n, 0.0) + jnp.log1p(jnp.exp(-jnp.abs(n)))
    out_ref[...] = g + sp * eps_ref[...]


def _compute_logits(x, wc, eps):
    return pl.pallas_call(
        _logits_body,
        grid=(_N_TOK // _BT,),
        in_specs=[
            pl.BlockSpec((_BT, _DIM), lambda i: (i, 0)),
            pl.BlockSpec((_DIM, 2 * _NE), lambda i: (0, 0)),
            pl.BlockSpec((_BT, _NE), lambda i: (i, 0)),
        ],
        out_specs=pl.BlockSpec((_BT, _NE), lambda i: (i, 0)),
        out_shape=jax.ShapeDtypeStruct((_N_TOK, _NE), jnp.float32),
    )(x, wc, eps)


def _route(logits_hbm, out_hbm, lg_v, w_v):
    nwords = _TPW * _NE
    wid = lax.axis_index("s") * 2 + lax.axis_index("c")
    base = wid * nwords
    pltpu.sync_copy(logits_hbm.at[pl.ds(base, nwords)], lg_v)

    lanes = lax.iota(jnp.int32, _L)
    zeros16 = jnp.zeros((_L,), jnp.float32)

    def zero_body(t, c):
        w_v[pl.ds(t * _L, _L)] = zeros16
        return c

    lax.fori_loop(0, nwords // _L, zero_body, 0)

    neg = jnp.full((_L,), -jnp.inf, jnp.float32)
    zi = jnp.zeros((_L,), jnp.int32)

    def group_body(g, c):
        tok64 = (g * _L + lanes) * _NE

        def exp_body(e, carry):
            m1, m2, i1, i2 = carry
            ev = jnp.full((_L,), e, jnp.int32)
            v = plsc.load_gather(lg_v, [tok64 + ev])
            gt1 = v > m1
            gt2 = v > m2
            i2n = jnp.where(gt1, i1, jnp.where(gt2, ev, i2))
            m2n = jnp.where(gt1, m1, jnp.where(gt2, v, m2))
            i1n = jnp.where(gt1, ev, i1)
            m1n = jnp.where(gt1, v, m1)
            return m1n, m2n, i1n, i2n

        m1, m2, i1, i2 = lax.fori_loop(0, _NE, exp_body, (neg, neg, zi, zi))
        e2 = jnp.exp(m2 - m1)
        denom = 1.0 + e2
        plsc.store_scatter(w_v, [tok64 + i1], 1.0 / denom)
        plsc.store_scatter(w_v, [tok64 + i2], e2 / denom)
        return c

    lax.fori_loop(0, _TPW // _L, group_body, 0)
    pltpu.sync_copy(w_v, out_hbm.at[pl.ds(base, nwords)])


@functools.cache
def _route_call():
    mesh = plsc.VectorSubcoreMesh(
        core_axis_name="c", subcore_axis_name="s", num_cores=2,
        num_subcores=16)
    return pl.kernel(
        _route,
        out_type=jax.ShapeDtypeStruct((_N_TOK * _NE,), jnp.float32),
        mesh=mesh,
        scratch_types=[
            pltpu.VMEM((_TPW * _NE,), jnp.float32),
            pltpu.VMEM((_TPW * _NE,), jnp.float32),
        ],
        compiler_params=pltpu.CompilerParams(needs_layout_passes=False),
    )


def _loss_body(w_ref, out_ref, acc_ref):
    i = pl.program_id(0)

    @pl.when(i == 0)
    def _():
        acc_ref[...] = jnp.zeros_like(acc_ref)

    acc_ref[...] += jnp.sum(w_ref[...], axis=0, keepdims=True)

    @pl.when(i == pl.num_programs(0) - 1)
    def _():
        imp = acc_ref[...]
        mean = jnp.sum(imp) / _NE
        var = jnp.sum((imp - mean) ** 2) / _NE
        out_ref[0, 0] = _UTIL * var / (mean * mean)


def _compute_loss(weights):
    return pl.pallas_call(
        _loss_body,
        grid=(_N_TOK // _BT,),
        in_specs=[pl.BlockSpec((_BT, _NE), lambda i: (i, 0))],
        out_specs=pl.BlockSpec(memory_space=pltpu.SMEM),
        out_shape=jax.ShapeDtypeStruct((1, 1), jnp.float32),
        scratch_shapes=[pltpu.VMEM((1, _NE), jnp.float32)],
    )(weights)


def kernel(x, Wg, Wnoise, noise_eps):
    wc = jnp.concatenate([Wg.T, Wnoise.T], axis=1)
    logits = _compute_logits(x, wc, noise_eps)
    weights = _route_call()(logits.reshape(-1)).reshape(_N_TOK, _NE)
    loss = _compute_loss(weights)[0, 0]
    return weights, loss


# speedup vs baseline: 2.2255x; 1.1424x over previous
import functools

import jax
import jax.numpy as jnp
from jax import lax
from jax.experimental import pallas as pl
from jax.experimental.pallas import tpu as pltpu
from jax.experimental.pallas import tpu_sc as plsc

_N_TOK = 16384
_DIM = 2048
_NE = 64
_UTIL = 0.01

_BT = 512
_NW = 32
_TPW = _N_TOK // _NW
_L = 16


def _logits_body(x_ref, wc_ref, eps_ref, out_ref):
    both = lax.dot_general(
        x_ref[...], wc_ref[...], (((1,), (0,)), ((), ())),
        preferred_element_type=jnp.float32)
    g = both[:, :_NE]
    n = both[:, _NE:]
    sp = jnp.maximum(n, 0.0) + jnp.log1p(jnp.exp(-jnp.abs(n)))
    out_ref[...] = g + sp * eps_ref[...]


def _compute_logits(x, wc, eps):
    return pl.pallas_call(
        _logits_body,
        grid=(_N_TOK // _BT,),
        in_specs=[
            pl.BlockSpec((_BT, _DIM), lambda i: (i, 0)),
            pl.BlockSpec((_DIM, 2 * _NE), lambda i: (0, 0)),
            pl.BlockSpec((_BT, _NE), lambda i: (i, 0)),
        ],
        out_specs=pl.BlockSpec((_BT, _NE), lambda i: (i, 0)),
        out_shape=jax.ShapeDtypeStruct((_N_TOK, _NE), jnp.float32),
    )(x, wc, eps)


def _route(logits_hbm, out_hbm, lg_v, w_v):
    nwords = _TPW * _NE
    wid = lax.axis_index("s") * 2 + lax.axis_index("c")
    row0 = wid * _TPW
    pltpu.sync_copy(logits_hbm.at[pl.ds(row0, _TPW)], lg_v)

    lanes = lax.iota(jnp.int32, _L)
    zeros16 = jnp.zeros((_L,), jnp.float32)

    def zero_body(t, c):
        for k2 in range(_NE // _L):
            w_v[t, pl.ds(k2 * _L, _L)] = zeros16
        return c

    lax.fori_loop(0, _TPW, zero_body, 0)

    neg = jnp.full((_L,), -jnp.inf, jnp.float32)
    zi = jnp.zeros((_L,), jnp.int32)

    def group_body(g, c):
        tok = g * _L + lanes

        def exp_body(e, carry):
            m1, m2, i1, i2 = carry
            ev = jnp.full((_L,), e, jnp.int32)
            v = plsc.load_gather(lg_v, [tok, ev])
            gt1 = v > m1
            gt2 = v > m2
            i2n = jnp.where(gt1, i1, jnp.where(gt2, ev, i2))
            m2n = jnp.where(gt1, m1, jnp.where(gt2, v, m2))
            i1n = jnp.where(gt1, ev, i1)
            m1n = jnp.where(gt1, v, m1)
            return m1n, m2n, i1n, i2n

        m1, m2, i1, i2 = lax.fori_loop(0, _NE, exp_body, (neg, neg, zi, zi))
        e2 = jnp.exp(m2 - m1)
        denom = 1.0 + e2
        plsc.store_scatter(w_v, [tok, i1], 1.0 / denom)
        plsc.store_scatter(w_v, [tok, i2], e2 / denom)
        return c

    lax.fori_loop(0, _TPW // _L, group_body, 0)
    pltpu.sync_copy(w_v, out_hbm.at[pl.ds(row0, _TPW)])


@functools.cache
def _route_call():
    mesh = plsc.VectorSubcoreMesh(
        core_axis_name="c", subcore_axis_name="s", num_cores=2,
        num_subcores=16)
    return pl.kernel(
        _route,
        out_type=jax.ShapeDtypeStruct((_N_TOK, _NE), jnp.float32),
        mesh=mesh,
        scratch_types=[
            pltpu.VMEM((_TPW, _NE), jnp.float32),
            pltpu.VMEM((_TPW, _NE), jnp.float32),
        ],
        compiler_params=pltpu.CompilerParams(needs_layout_passes=False),
    )


def _loss_body(w_ref, out_ref, acc_ref):
    i = pl.program_id(0)

    @pl.when(i == 0)
    def _():
        acc_ref[...] = jnp.zeros_like(acc_ref)

    acc_ref[...] += jnp.sum(w_ref[...], axis=0, keepdims=True)

    @pl.when(i == pl.num_programs(0) - 1)
    def _():
        imp = acc_ref[...]
        mean = jnp.sum(imp) / _NE
        var = jnp.sum((imp - mean) ** 2) / _NE
        out_ref[0, 0] = _UTIL * var / (mean * mean)


def _compute_loss(weights):
    return pl.pallas_call(
        _loss_body,
        grid=(_N_TOK // _BT,),
        in_specs=[pl.BlockSpec((_BT, _NE), lambda i: (i, 0))],
        out_specs=pl.BlockSpec(memory_space=pltpu.SMEM),
        out_shape=jax.ShapeDtypeStruct((1, 1), jnp.float32),
        scratch_shapes=[pltpu.VMEM((1, _NE), jnp.float32)],
    )(weights)


def kernel(x, Wg, Wnoise, noise_eps):
    wc = jnp.concatenate([Wg.T, Wnoise.T], axis=1)
    logits = _compute_logits(x, wc, noise_eps)
    weights = _route_call()(logits)
    loss = _compute_loss(weights)[0, 0]
    return weights, loss
